# baseline (speedup 1.0000x reference)
import jax
import jax.numpy as jnp
from jax import lax
from jax.experimental import pallas as pl
from jax.experimental.pallas import tpu as pltpu
from jax.experimental.pallas import tpu_sc as plsc

_NUM_CORES = 2
_NUM_SUBCORES = 16
_NUM_WORKERS = _NUM_CORES * _NUM_SUBCORES
_LANES = 16

_IDX_W = 128
_U = 2
_STEP_ROWS = _U * _IDX_W
_PITCH = _IDX_W + 1


def _gather_body(nsteps, dim, x_hbm, tab_hbm, out_hbm,
                 ib0, ib1, gb0, gb1, tb0, tb1,
                 isem, gsem0, gsem1, osem0, osem1):
  scale = dim ** 0.5
  ndt = dim // 8
  steps_per_slab = _IDX_W // _U

  wid = lax.axis_index("s") * _NUM_CORES + lax.axis_index("c")
  step0 = wid * nsteps

  ibufs = (ib0, ib1)
  gbufs = (gb0, gb1)
  tbufs = (tb0, tb1)
  gsems = (gsem0, gsem1)
  osems = (osem0, osem1)

  iot = lax.iota(jnp.int32, _LANES)
  dtvs = [(k * _LANES + iot) >> 3 for k in range(dim // _LANES)]
  divs = [(k * _LANES + iot) & 7 for k in range(dim // _LANES)]

  def idx_load(u, p):
    s = u // steps_per_slab
    bt0 = (u % steps_per_slab) * _U
    pltpu.async_copy(
        x_hbm.at[pl.ds(s * _IDX_W + bt0, _U)], ibufs[p], isem).wait()

  def gather_start(p):
    for j in range(_U):
      pltpu.async_copy(
          tab_hbm.at[ibufs[p].at[j]],
          gbufs[p].at[pl.ds(j * _IDX_W, _IDX_W)],
          gsems[p])

  def gather_wait(p):
    for j in range(_U):
      pltpu.make_async_copy(
          tab_hbm.at[ibufs[p].at[j]],
          gbufs[p].at[pl.ds(j * _IDX_W, _IDX_W)],
          gsems[p]).wait()

  def out_copies(u, p):
    s = u // steps_per_slab
    bt0 = (u % steps_per_slab) * _U
    for dt in range(ndt):
      yield (tbufs[p].at[dt, :, :, pl.ds(0, _IDX_W)],
             out_hbm.at[s * ndt + dt, pl.ds(bt0, _U)])

  def out_start(u, p):
    for src, dst in out_copies(u, p):
      pltpu.async_copy(src, dst, osems[p])

  def out_wait(u, p):
    for src, dst in out_copies(u, p):
      pltpu.make_async_copy(src, dst, osems[p]).wait()

  def transpose_scale(p):
    gbuf = gbufs[p]
    tbuf = tbufs[p]

    @plsc.parallel_loop(0, _STEP_ROWS, unroll=2)
    def _tr(r):
      btlv = jnp.full((_LANES,), r >> 7, jnp.int32)
      rv = jnp.full((_LANES,), r & (_IDX_W - 1), jnp.int32)
      for k in range(dim // _LANES):
        v = gbuf[r, pl.ds(k * _LANES, _LANES)] * scale
        plsc.store_scatter(tbuf, [dtvs[k], btlv, divs[k], rv], v)

  for p in range(2):
    idx_load(step0 + p, p)
    gather_start(p)

  @pl.loop(0, nsteps, step=2)
  def _steady(i0):
    for p in range(2):
      i = i0 + p
      u = step0 + i
      gather_wait(p)

      @pl.when(i >= 2)
      def _():
        out_wait(u - 2, p)

      transpose_scale(p)

      @pl.when(i + 2 < nsteps)
      def _():
        idx_load(u + 2, p)
        gather_start(p)

      out_start(u, p)

  for i in (nsteps - 2, nsteps - 1):
    out_wait(step0 + i, i % 2)


def kernel(x, lut):
  batch, seq = x.shape
  vocab, dim = lut.shape
  n = x.size
  assert batch % (_IDX_W * _U) == 0 and dim % 8 == 0
  nsteps_total = n // _STEP_ROWS
  assert nsteps_total % _NUM_WORKERS == 0
  nsteps = nsteps_total // _NUM_WORKERS
  assert nsteps % 2 == 0
  ndt = dim // 8

  xs = jnp.transpose(x).astype(jnp.int32).reshape(seq * (batch // _IDX_W),
                                                  _IDX_W)

  mesh = plsc.VectorSubcoreMesh(
      core_axis_name="c", subcore_axis_name="s",
      num_cores=_NUM_CORES, num_subcores=_NUM_SUBCORES)
  run = pl.kernel(
      lambda *refs: _gather_body(nsteps, dim, *refs),
      out_type=jax.ShapeDtypeStruct(
          (seq * ndt, batch // _IDX_W, 8, _IDX_W), jnp.float32),
      mesh=mesh,
      scratch_types=(
          [pltpu.VMEM((_U, _IDX_W), jnp.int32) for _ in range(2)]
          + [pltpu.VMEM((_STEP_ROWS, dim), jnp.float32) for _ in range(2)]
          + [pltpu.VMEM((ndt, _U, 8, _PITCH), jnp.float32) for _ in range(2)]
          + [pltpu.SemaphoreType.DMA] * 5
      ),
      compiler_params=pltpu.CompilerParams(use_tc_tiling_on_sc=False,
                                           needs_layout_passes=False),
      name="sc_embedding_lookup",
  )
  out5 = run(xs, lut)
  out = out5.reshape(seq, ndt, batch // _IDX_W, 8, _IDX_W)
  return out.transpose(2, 4, 0, 1, 3).reshape(batch, seq, dim)


# speedup vs baseline: 1.0004x; 1.0004x over previous
import jax
import jax.numpy as jnp
from jax import lax
from jax.experimental import pallas as pl
from jax.experimental.pallas import tpu as pltpu
from jax.experimental.pallas import tpu_sc as plsc

_NUM_CORES = 2
_NUM_SUBCORES = 16
_NUM_WORKERS = _NUM_CORES * _NUM_SUBCORES
_LANES = 16

_IDX_W = 128
_U = 2
_STEP_ROWS = _U * _IDX_W
_PITCH = _IDX_W + 1


def _gather_body(nsteps, dim, x_hbm, tab_hbm, out_hbm,
                 ib0, ib1, gb0, gb1, tb0, tb1,
                 isem, gsem0, gsem1, osem0, osem1):
  scale = dim ** 0.5
  ndt = dim // 8
  steps_per_slab = _IDX_W // _U

  wid = lax.axis_index("s") * _NUM_CORES + lax.axis_index("c")
  step0 = wid * nsteps

  ibufs = (ib0, ib1)
  gbufs = (gb0, gb1)
  tbufs = (tb0, tb1)
  gsems = (gsem0, gsem1)
  osems = (osem0, osem1)

  iot = lax.iota(jnp.int32, _LANES)
  dtvs = [(k * _LANES + iot) >> 3 for k in range(dim // _LANES)]
  divs = [(k * _LANES + iot) & 7 for k in range(dim // _LANES)]

  def idx_load(u, p):
    s = u // steps_per_slab
    bt0 = (u % steps_per_slab) * _U
    pltpu.async_copy(
        x_hbm.at[pl.ds(s * _IDX_W + bt0, _U)], ibufs[p], isem).wait()

  def gather_start(p):
    for j in range(_U):
      pltpu.async_copy(
          tab_hbm.at[ibufs[p].at[j]],
          gbufs[p].at[pl.ds(j * _IDX_W, _IDX_W)],
          gsems[p])

  def gather_wait(p):
    for j in range(_U):
      pltpu.make_async_copy(
          tab_hbm.at[ibufs[p].at[j]],
          gbufs[p].at[pl.ds(j * _IDX_W, _IDX_W)],
          gsems[p]).wait()

  def out_copies(u, p):
    s = u // steps_per_slab
    bt0 = (u % steps_per_slab) * _U
    for dt in range(ndt):
      yield (tbufs[p].at[dt, pl.ds(0, _U), :, pl.ds(0, _IDX_W)],
             out_hbm.at[s * ndt + dt, pl.ds(bt0, _U)])

  def out_start(u, p):
    for src, dst in out_copies(u, p):
      pltpu.async_copy(src, dst, osems[p])

  def out_wait(u, p):
    for src, dst in out_copies(u, p):
      pltpu.make_async_copy(src, dst, osems[p]).wait()

  def transpose_scale(p):
    gbuf = gbufs[p]
    tbuf = tbufs[p]

    @plsc.parallel_loop(0, _STEP_ROWS, unroll=2)
    def _tr(r):
      btlv = jnp.full((_LANES,), r >> 7, jnp.int32)
      rv = jnp.full((_LANES,), r & (_IDX_W - 1), jnp.int32)
      for k in range(dim // _LANES):
        v = gbuf[r, pl.ds(k * _LANES, _LANES)] * scale
        plsc.store_scatter(tbuf, [dtvs[k], btlv, divs[k], rv], v)

  for p in range(2):
    idx_load(step0 + p, p)
    gather_start(p)

  @pl.loop(0, nsteps, step=2)
  def _steady(i0):
    for p in range(2):
      i = i0 + p
      u = step0 + i
      gather_wait(p)

      @pl.when(i >= 2)
      def _():
        out_wait(u - 2, p)

      transpose_scale(p)

      @pl.when(i + 2 < nsteps)
      def _():
        idx_load(u + 2, p)
        gather_start(p)

      out_start(u, p)

  for i in (nsteps - 2, nsteps - 1):
    out_wait(step0 + i, i % 2)


def kernel(x, lut):
  batch, seq = x.shape
  vocab, dim = lut.shape
  n = x.size
  assert batch % (_IDX_W * _U) == 0 and dim % 8 == 0
  nsteps_total = n // _STEP_ROWS
  assert nsteps_total % _NUM_WORKERS == 0
  nsteps = nsteps_total // _NUM_WORKERS
  assert nsteps % 2 == 0
  ndt = dim // 8

  xs = jnp.transpose(x).astype(jnp.int32).reshape(seq * (batch // _IDX_W),
                                                  _IDX_W)

  mesh = plsc.VectorSubcoreMesh(
      core_axis_name="c", subcore_axis_name="s",
      num_cores=_NUM_CORES, num_subcores=_NUM_SUBCORES)
  run = pl.kernel(
      lambda *refs: _gather_body(nsteps, dim, *refs),
      out_type=jax.ShapeDtypeStruct(
          (seq * ndt, batch // _IDX_W, 8, _IDX_W), jnp.float32),
      mesh=mesh,
      scratch_types=(
          [pltpu.VMEM((_U, _IDX_W), jnp.int32) for _ in range(2)]
          + [pltpu.VMEM((_STEP_ROWS, dim), jnp.float32) for _ in range(2)]
          + [pltpu.VMEM((ndt, _U + 1, 8, _PITCH), jnp.float32)
             for _ in range(2)]
          + [pltpu.SemaphoreType.DMA] * 5
      ),
      compiler_params=pltpu.CompilerParams(use_tc_tiling_on_sc=False,
                                           needs_layout_passes=False),
      name="sc_embedding_lookup",
  )
  out5 = run(xs, lut)
  out = out5.reshape(seq, ndt, batch // _IDX_W, 8, _IDX_W)
  return out.transpose(2, 4, 0, 1, 3).reshape(batch, seq, dim)
